# baseline (speedup 1.0000x reference)
import functools
import jax
import jax.numpy as jnp
from jax import lax
from jax.experimental import pallas as pl
from jax.experimental.pallas import tpu as pltpu
from jax.experimental.pallas import tpu_sc as plsc

_ROWS = 4096
_COLS = 2048
_NW = 32
_NS = 16
_A_CH = _ROWS // _NW
_B_CH = 2 * _ROWS // _NW
_CH = 8
_K = (_A_CH + _B_CH) // _CH
_KA = _A_CH // _CH
_NB = 3
_NP = 2
_ST = _NP * _NB

_mesh = plsc.VectorSubcoreMesh(core_axis_name="c", subcore_axis_name="s")


@functools.partial(
    pl.kernel,
    mesh=_mesh,
    out_type=jax.ShapeDtypeStruct((3 * _ROWS, _COLS), jnp.float32),
    scratch_types=(
        [pltpu.VMEM((_CH, _COLS), jnp.float32)] * _NB
        + [pltpu.VMEM_SHARED((_NS, _NB, _CH, _COLS), jnp.float32)]
        + [pltpu.SemaphoreType.DMA] * (2 * _ST)
    ),
)
def _sc_copy(x_hbm, o_hbm, *scratch):
    tbufs = scratch[:_NB]
    shared = scratch[_NB]
    isems = scratch[_NB + 1:_NB + 1 + _ST]
    osems = scratch[_NB + 1 + _ST:]
    sid = lax.axis_index("s")
    wid = lax.axis_index("c") * _NS + sid
    a0 = wid * _A_CH
    b_dst = _ROWS + wid * _B_CH
    b_src = 2 * _ROWS + wid * _B_CH

    def src_row(c):
        off = c * _CH
        if c < _KA:
            return a0 + off
        return b_src + (off - _A_CH)

    def dst_row(c):
        off = c * _CH
        if c < _KA:
            return a0 + off
        return b_dst + (off - _A_CH)

    def buf(c):
        p, s = c % _NP, (c // _NP) % _NB
        if p == 0:
            return tbufs[s]
        return shared.at[sid, s]

    def in_copy(c):
        return pltpu.async_copy(
            x_hbm.at[pl.ds(src_row(c), _CH)], buf(c), isems[c % _ST]
        )

    def out_copy(c):
        return pltpu.async_copy(
            buf(c), o_hbm.at[pl.ds(dst_row(c), _CH)], osems[c % _ST]
        )

    h_in = {c: in_copy(c) for c in range(min(_ST, _K))}
    h_out = {}
    for c in range(_K):
        h_in[c].wait()
        h_out[c] = out_copy(c)
        if c + _ST < _K:
            h_out[c].wait()
            h_in[c + _ST] = in_copy(c + _ST)
    for c in range(max(0, _K - _ST), _K):
        h_out[c].wait()


def kernel(x):
    x2 = x.reshape(5 * _ROWS, _COLS)
    out = _sc_copy(x2)
    return out.reshape(3, _ROWS, _COLS)
